# Initial kernel scaffold; baseline (speedup 1.0000x reference)
import jax
import jax.numpy as jnp
from jax.experimental import pallas as pl


def kernel(student_output, teacher_output):
    raise NotImplementedError("write your pallas kernel here")


# speedup vs baseline: 18.2364x; 18.2364x over previous
import functools

import jax
import jax.numpy as jnp
from jax.experimental import pallas as pl
from jax.experimental.pallas import tpu as pltpu

K = 5
EPS = 1e-8
N = 4096
D = 128
BR = 256


def _knn_loss_kernel(s_ref, t_ref, out_ref, acc_ref):
    i = pl.program_id(0)
    nsteps = pl.num_programs(0)

    s = s_ref[...]
    t = t_ref[...]
    x = jax.lax.dot_general(
        s, t, (((1,), (1,)), ((), ())), preferred_element_type=jnp.float32
    )

    neg = jnp.float32(-3.0e38)
    thr = jnp.full((BR, 1), jnp.float32(3.0e38))
    remaining = jnp.full((BR, 1), jnp.float32(K))
    s_top = jnp.zeros((BR, 1), jnp.float32)
    for _ in range(K):
        y = jnp.where(x < thr, x, neg)
        m = jnp.max(y, axis=1, keepdims=True)
        c = jnp.sum(jnp.where(x == m, jnp.float32(1.0), jnp.float32(0.0)),
                    axis=1, keepdims=True)
        take = jnp.minimum(c, remaining)
        s_top = s_top + take * m
        remaining = remaining - take
        thr = m

    mean5 = s_top * jnp.float32(1.0 / K)
    partial = jnp.sum(jnp.log(mean5 + jnp.float32(EPS)))

    @pl.when(i == 0)
    def _init():
        acc_ref[0] = jnp.float32(0.0)

    acc_ref[0] = acc_ref[0] + partial

    @pl.when(i == nsteps - 1)
    def _fin():
        out_ref[0] = -acc_ref[0] * jnp.float32(1.0 / N)


@functools.partial(jax.jit, static_argnames=("interpret",))
def kernel(student_output, teacher_output, interpret=False):
    nsteps = N // BR
    out = pl.pallas_call(
        _knn_loss_kernel,
        grid=(nsteps,),
        in_specs=[
            pl.BlockSpec((BR, D), lambda i: (i, 0)),
            pl.BlockSpec((N, D), lambda i: (0, 0)),
        ],
        out_specs=pl.BlockSpec(memory_space=pltpu.SMEM),
        out_shape=jax.ShapeDtypeStruct((1,), jnp.float32),
        scratch_shapes=[pltpu.SMEM((1,), jnp.float32)],
        interpret=interpret,
    )(student_output, teacher_output)
    return jnp.reshape(out, ())
